# baseline (speedup 1.0000x reference)
import jax
import jax.numpy as jnp
from jax.experimental import pallas as pl
from jax.experimental.pallas import tpu as pltpu

_N = 4096
_D = 128
_BM = 512
_NI = _N // _BM


def _body(x_ref, wi_ref, ws_ref, l_ref, u_ref, out_ref, hi_ref, hs_ref):
    i = pl.program_id(0)

    @pl.when(i == 0)
    def _():
        hi_ref[...] = jnp.dot(x_ref[...], wi_ref[...],
                              preferred_element_type=jnp.float32)
        hs_ref[...] = jnp.dot(x_ref[...], ws_ref[...],
                              preferred_element_type=jnp.float32)

    out_ref[...] = jnp.maximum(
        jnp.dot(l_ref[...], hi_ref[...], preferred_element_type=jnp.float32)
        + jnp.dot(u_ref[...], hs_ref[...], preferred_element_type=jnp.float32),
        0.0)


def kernel(x, lower_neighborhood, upper_neighborhood, W_irr, W_sol):
    return pl.pallas_call(
        _body,
        grid=(_NI,),
        in_specs=[
            pl.BlockSpec((_N, _D), lambda i: (0, 0)),
            pl.BlockSpec((_D, _D), lambda i: (0, 0)),
            pl.BlockSpec((_D, _D), lambda i: (0, 0)),
            pl.BlockSpec((_BM, _N), lambda i: (i, 0)),
            pl.BlockSpec((_BM, _N), lambda i: (i, 0)),
        ],
        out_specs=pl.BlockSpec((_BM, _D), lambda i: (i, 0)),
        out_shape=jax.ShapeDtypeStruct((_N, _D), jnp.float32),
        scratch_shapes=[
            pltpu.VMEM((_N, _D), jnp.float32),
            pltpu.VMEM((_N, _D), jnp.float32),
        ],
        compiler_params=pltpu.CompilerParams(
            dimension_semantics=("arbitrary",)),
    )(x, W_irr, W_sol, lower_neighborhood, upper_neighborhood)


# speedup vs baseline: 1.0350x; 1.0350x over previous
import jax
import jax.numpy as jnp
from jax.experimental import pallas as pl
from jax.experimental.pallas import tpu as pltpu

_N = 4096
_D = 128
_BM = 256
_NI = _N // _BM


def _body(x_ref, wi_ref, ws_ref, l_ref, u_ref, out_ref, hi_ref, hs_ref):
    i = pl.program_id(0)

    @pl.when(i == 0)
    def _():
        hi_ref[...] = jnp.dot(x_ref[...], wi_ref[...],
                              preferred_element_type=jnp.float32)
        hs_ref[...] = jnp.dot(x_ref[...], ws_ref[...],
                              preferred_element_type=jnp.float32)

    out_ref[...] = jnp.maximum(
        jnp.dot(l_ref[...], hi_ref[...], preferred_element_type=jnp.float32)
        + jnp.dot(u_ref[...], hs_ref[...], preferred_element_type=jnp.float32),
        0.0)


def kernel(x, lower_neighborhood, upper_neighborhood, W_irr, W_sol):
    return pl.pallas_call(
        _body,
        grid=(_NI,),
        in_specs=[
            pl.BlockSpec((_N, _D), lambda i: (0, 0)),
            pl.BlockSpec((_D, _D), lambda i: (0, 0)),
            pl.BlockSpec((_D, _D), lambda i: (0, 0)),
            pl.BlockSpec((_BM, _N), lambda i: (i, 0)),
            pl.BlockSpec((_BM, _N), lambda i: (i, 0)),
        ],
        out_specs=pl.BlockSpec((_BM, _D), lambda i: (i, 0)),
        out_shape=jax.ShapeDtypeStruct((_N, _D), jnp.float32),
        scratch_shapes=[
            pltpu.VMEM((_N, _D), jnp.float32),
            pltpu.VMEM((_N, _D), jnp.float32),
        ],
        compiler_params=pltpu.CompilerParams(
            dimension_semantics=("arbitrary",)),
    )(x, W_irr, W_sol, lower_neighborhood, upper_neighborhood)
